# baseline (speedup 1.0000x reference)
import dataclasses
import functools

import jax
import jax.numpy as jnp
from jax import lax
from jax.experimental import pallas as pl
from jax.experimental.pallas import tpu as pltpu
from jax.experimental.pallas import tpu_sc as plsc

N = 10000
E = 320000
IN = 128
HID = 256
OUT = 128

NC = 2
NS = 16
NW = NC * NS
CH = 128
NCHUNK = 160
GS = 8
NG = NCHUNK // GS
EPW = CH * NCHUNK
EPAD = NS * EPW
DH = 64
NP = 10112
RPS = NP // NS
D = 128

_sc_mesh = plsc.VectorSubcoreMesh(core_axis_name="c", subcore_axis_name="s")


def _sc_agg_body(with_counts, *refs):
    if with_counts:
        (feat2, srcp, dstp, part, cnt_out,
         acc_sh, feat_sh, src_h, dst_h, rows0, rows1, cnt_v,
         sem0, sem1, isem) = refs
    else:
        (feat2, srcp, dstp, part,
         acc_sh, feat_sh, src_h, dst_h, rows0, rows1,
         sem0, sem1, isem) = refs
        cnt_v = cnt_out = None

    cid = lax.axis_index("c")
    sid = lax.axis_index("s")
    wid = sid * NC + cid

    base = sid * RPS
    sl = pl.ds(base, RPS)
    tail = RPS % CH

    pltpu.sync_copy(feat2.at[cid].at[sl], feat_sh.at[sl])

    zvec = jnp.zeros((16,), jnp.float32)

    @pl.loop(0, CH)
    def _(r):
        @pl.loop(0, DH, step=16)
        def _(j):
            rows0[r, pl.ds(j, 16)] = zvec

    if cnt_v is not None:
        @pl.loop(0, NP, step=16)
        def _(j):
            cnt_v[pl.ds(j, 16)] = zvec

    @pl.loop(0, RPS - CH + 1, step=CH)
    def _(r0):
        pltpu.sync_copy(rows0, acc_sh.at[pl.ds(base + r0, CH)])

    if tail:
        pltpu.sync_copy(rows0.at[pl.ds(0, tail)],
                        acc_sh.at[pl.ds(base + RPS - tail, tail)])

    plsc.subcore_barrier()

    ones = jnp.ones((16,), jnp.float32)

    def idx_dma(g, h):
        gsl = pl.ds(g * GS, GS)
        pltpu.async_copy(srcp.at[sid].at[gsl], src_h.at[h], isem)
        pltpu.async_copy(dstp.at[sid].at[gsl], dst_h.at[h], isem)

    def idx_wait(h):
        pltpu.make_async_copy(srcp.at[sid].at[pl.ds(0, GS)],
                              src_h.at[h], isem).wait()
        pltpu.make_async_copy(dstp.at[sid].at[pl.ds(0, GS)],
                              dst_h.at[h], isem).wait()

    def gather_start(h, i, rows, sem):
        pltpu.async_copy(feat_sh.at[src_h.at[h, i]], rows, sem)

    def gather_wait(h, i, rows, sem):
        pltpu.make_async_copy(feat_sh.at[src_h.at[h, i]], rows, sem).wait()

    def chunk_work(h, i, rows, sem):
        gather_wait(h, i, rows, sem)
        pltpu.sync_copy(rows, acc_sh.at[dst_h.at[h, i]], add=True)
        if cnt_v is not None:
            for j in range(0, CH, 16):
                plsc.addupdate_scatter(
                    cnt_v, [dst_h[h, i, pl.ds(j, 16)]], ones)

    rb = (rows0, rows1)
    sb = (sem0, sem1)

    idx_dma(0, 0)
    idx_dma(1, 1)
    idx_wait(0)
    gather_start(0, 0, rows0, sem0)

    def group(g, h, nh, prefetch_group, last):
        for i in range(GS):
            if i < GS - 1:
                if i == GS - 2 and not last:
                    idx_wait(nh)
                gather_start(h, i + 1, rb[(i + 1) % 2], sb[(i + 1) % 2])
            elif not last:
                gather_start(nh, 0, rb[0], sb[0])
            chunk_work(h, i, rb[i % 2], sb[i % 2])
        if prefetch_group is not None:
            idx_dma(prefetch_group, h)

    @pl.loop(0, NG - 2)
    def _(g):
        h = lax.rem(g, 2)
        group(g, h, 1 - h, g + 2, last=False)

    hl = (NG - 2) % 2
    group(NG - 2, hl, 1 - hl, None, last=False)
    group(NG - 1, 1 - hl, hl, None, last=True)

    plsc.subcore_barrier()

    pltpu.sync_copy(acc_sh.at[sl], part.at[cid].at[sl])
    if cnt_v is not None:
        pltpu.sync_copy(cnt_v, cnt_out.at[wid])


def _make_sc_agg(with_counts):
    cp = pltpu.CompilerParams()
    if "needs_layout_passes" in pltpu.CompilerParams.__dataclass_fields__:
        cp = dataclasses.replace(cp, needs_layout_passes=False)
    out_type = [jax.ShapeDtypeStruct((NC, NP, DH), jnp.float32)]
    scratch = [
        pltpu.VMEM_SHARED((NP, DH), jnp.float32),
        pltpu.VMEM_SHARED((NP, DH), jnp.float32),
        pltpu.VMEM((2, GS, CH), jnp.int32),
        pltpu.VMEM((2, GS, CH), jnp.int32),
        pltpu.VMEM((CH, DH), jnp.float32),
        pltpu.VMEM((CH, DH), jnp.float32),
    ]
    if with_counts:
        out_type.append(jax.ShapeDtypeStruct((NW, NP), jnp.float32))
        scratch.append(pltpu.VMEM((NP,), jnp.float32))
    scratch.append(pltpu.SemaphoreType.DMA)
    scratch.append(pltpu.SemaphoreType.DMA)
    scratch.append(pltpu.SemaphoreType.DMA)
    return pl.kernel(
        functools.partial(_sc_agg_body, with_counts),
        out_type=tuple(out_type),
        mesh=_sc_mesh,
        scratch_types=scratch,
        compiler_params=cp,
    )


_sc_agg1 = _make_sc_agg(True)
_sc_agg2 = _make_sc_agg(False)


BLK = 1264


def _dot_t(a, w):
    return jax.lax.dot_general(
        a.astype(jnp.bfloat16), w.astype(jnp.bfloat16),
        (((1,), (1,)), ((), ())), preferred_element_type=jnp.float32)


BLK = NP // 8


def _tc_layer1_body(part, cnt2, x, W1l, b1, W1r, W2l, W2r, b2,
                    p_out, q_out, inv_out):
    agg = jnp.concatenate([part[0], part[1]], axis=1)
    inv = 1.0 / jnp.maximum(cnt2[...], 1.0)
    mean = agg * inv
    h = _dot_t(mean, W1l[...]) + b1[...] + _dot_t(x[...], W1r[...])
    h = jnp.maximum(h, 0.0)
    p_out[0] = _dot_t(h, W2l[...][:DH])
    p_out[1] = _dot_t(h, W2l[...][DH:])
    q_out[...] = _dot_t(h, W2r[...]) + b2[...]
    inv_out[...] = inv


def _tc_layer1(part, cnt2, x, W1l, b1, W1r, W2l, b2, W2r):
    full = lambda i: (0, 0)
    return pl.pallas_call(
        _tc_layer1_body,
        grid=(NP // BLK,),
        in_specs=[
            pl.BlockSpec((NC, BLK, DH), lambda i: (0, i, 0)),
            pl.BlockSpec((BLK, 1), lambda i: (i, 0)),
            pl.BlockSpec((BLK, IN), lambda i: (i, 0)),
            pl.BlockSpec((HID, IN), full),
            pl.BlockSpec((HID,), lambda i: (0,)),
            pl.BlockSpec((HID, IN), full),
            pl.BlockSpec((OUT, HID), full),
            pl.BlockSpec((OUT, HID), full),
            pl.BlockSpec((OUT,), lambda i: (0,)),
        ],
        out_specs=[
            pl.BlockSpec((NC, BLK, DH), lambda i: (0, i, 0)),
            pl.BlockSpec((BLK, OUT), lambda i: (i, 0)),
            pl.BlockSpec((BLK, 1), lambda i: (i, 0)),
        ],
        out_shape=[
            jax.ShapeDtypeStruct((NC, NP, DH), jnp.float32),
            jax.ShapeDtypeStruct((NP, OUT), jnp.float32),
            jax.ShapeDtypeStruct((NP, 1), jnp.float32),
        ],
    )(part, cnt2, x, W1l, b1, W1r, W2l, W2r, b2)


def _tc_final_body(part, inv, q, out):
    agg = jnp.concatenate([part[0], part[1]], axis=1)
    out[...] = agg * inv[...] + q[...]


def _tc_final(part, inv, q):
    return pl.pallas_call(
        _tc_final_body,
        grid=(NP // BLK,),
        in_specs=[
            pl.BlockSpec((NC, BLK, DH), lambda i: (0, i, 0)),
            pl.BlockSpec((BLK, 1), lambda i: (i, 0)),
            pl.BlockSpec((BLK, OUT), lambda i: (i, 0)),
        ],
        out_specs=pl.BlockSpec((BLK, OUT), lambda i: (i, 0)),
        out_shape=jax.ShapeDtypeStruct((NP, OUT), jnp.float32),
    )(part, inv, q)


@jax.jit
def _run(x, edge_index, W1l, b1, W1r, W2l, b2, W2r):
    src = edge_index[0]
    dst = edge_index[1]
    srcp = jnp.pad(src, (0, EPAD - E)).reshape(NS, NCHUNK, CH)
    dstp = jnp.pad(dst, (0, EPAD - E),
                   constant_values=N).reshape(NS, NCHUNK, CH)
    xp = jnp.pad(x, ((0, NP - N), (0, 0)))
    x2 = jnp.stack([xp[:, :DH], xp[:, DH:]])

    part1, cntp = _sc_agg1(x2, srcp, dstp)
    cnt2 = (jnp.sum(cntp, axis=0) * 0.5)[:, None]
    p2, q, inv = _tc_layer1(part1, cnt2, xp, W1l, b1, W1r, W2l, b2, W2r)
    (part2,) = _sc_agg2(p2, srcp, dstp)
    return _tc_final(part2, inv, q)[:N]


def kernel(x, edge_index, W1l, b1, W1r, W2l, b2, W2r):
    return _run(x, edge_index, W1l, b1, W1r, W2l, b2, W2r)


# speedup vs baseline: 1.0261x; 1.0261x over previous
import dataclasses
import functools

import jax
import jax.numpy as jnp
from jax import lax
from jax.experimental import pallas as pl
from jax.experimental.pallas import tpu as pltpu
from jax.experimental.pallas import tpu_sc as plsc

N = 10000
E = 320000
IN = 128
HID = 256
OUT = 128

NC = 2
NS = 16
NW = NC * NS
CH = 128
NCHUNK = 160
GS = 8
NG = NCHUNK // GS
EPW = CH * NCHUNK
EPAD = NS * EPW
DH = 64
NP = 10112
RPS = NP // NS
D = 128

_sc_mesh = plsc.VectorSubcoreMesh(core_axis_name="c", subcore_axis_name="s")


def _sc_agg_body(with_counts, *refs):
    if with_counts:
        (feat2, srcp, dstp, part, cnt_out,
         acc_sh, feat_sh, src_h, dst_h, rows0, rows1, cnt_v,
         sem0, sem1, isem) = refs
    else:
        (feat2, srcp, dstp, part,
         acc_sh, feat_sh, src_h, dst_h, rows0, rows1,
         sem0, sem1, isem) = refs
        cnt_v = cnt_out = None

    cid = lax.axis_index("c")
    sid = lax.axis_index("s")
    wid = sid * NC + cid

    base = sid * RPS
    sl = pl.ds(base, RPS)
    tail = RPS % CH

    pltpu.sync_copy(feat2.at[cid].at[sl], feat_sh.at[sl])

    zvec = jnp.zeros((16,), jnp.float32)

    @pl.loop(0, CH)
    def _(r):
        @pl.loop(0, DH, step=16)
        def _(j):
            rows0[r, pl.ds(j, 16)] = zvec

    if cnt_v is not None:
        @pl.loop(0, NP, step=16)
        def _(j):
            cnt_v[pl.ds(j, 16)] = zvec

    @pl.loop(0, RPS - CH + 1, step=CH)
    def _(r0):
        pltpu.sync_copy(rows0, acc_sh.at[pl.ds(base + r0, CH)])

    if tail:
        pltpu.sync_copy(rows0.at[pl.ds(0, tail)],
                        acc_sh.at[pl.ds(base + RPS - tail, tail)])

    plsc.subcore_barrier()

    ones = jnp.ones((16,), jnp.float32)

    def idx_dma(g, h):
        gsl = pl.ds(g * GS, GS)
        pltpu.async_copy(srcp.at[sid].at[gsl], src_h.at[h], isem)
        pltpu.async_copy(dstp.at[sid].at[gsl], dst_h.at[h], isem)

    def idx_wait(h):
        pltpu.make_async_copy(srcp.at[sid].at[pl.ds(0, GS)],
                              src_h.at[h], isem).wait()
        pltpu.make_async_copy(dstp.at[sid].at[pl.ds(0, GS)],
                              dst_h.at[h], isem).wait()

    def gather_start(h, i, rows, sem):
        pltpu.async_copy(feat_sh.at[src_h.at[h, i]], rows, sem)

    def gather_wait(h, i, rows, sem):
        pltpu.make_async_copy(feat_sh.at[src_h.at[h, i]], rows, sem).wait()

    def chunk_work(h, i, rows, sem):
        gather_wait(h, i, rows, sem)
        pltpu.sync_copy(rows, acc_sh.at[dst_h.at[h, i]], add=True)
        if cnt_v is not None:
            for j in range(0, CH, 16):
                plsc.addupdate_scatter(
                    cnt_v, [dst_h[h, i, pl.ds(j, 16)]], ones)

    rb = (rows0, rows1)
    sb = (sem0, sem1)

    idx_dma(0, 0)
    idx_dma(1, 1)
    idx_wait(0)
    gather_start(0, 0, rows0, sem0)

    def group(g, h, nh, prefetch_group, last):
        for i in range(GS):
            if i < GS - 1:
                if i == GS - 2 and not last:
                    idx_wait(nh)
                gather_start(h, i + 1, rb[(i + 1) % 2], sb[(i + 1) % 2])
            elif not last:
                gather_start(nh, 0, rb[0], sb[0])
            chunk_work(h, i, rb[i % 2], sb[i % 2])
        if prefetch_group is not None:
            idx_dma(prefetch_group, h)

    @pl.loop(0, NG - 2)
    def _(g):
        h = lax.rem(g, 2)
        group(g, h, 1 - h, g + 2, last=False)

    hl = (NG - 2) % 2
    group(NG - 2, hl, 1 - hl, None, last=False)
    group(NG - 1, 1 - hl, hl, None, last=True)

    plsc.subcore_barrier()

    pltpu.sync_copy(acc_sh.at[sl], part.at[cid].at[sl])
    if cnt_v is not None:
        pltpu.sync_copy(cnt_v, cnt_out.at[wid])


def _make_sc_agg(with_counts):
    cp = pltpu.CompilerParams()
    if "needs_layout_passes" in pltpu.CompilerParams.__dataclass_fields__:
        cp = dataclasses.replace(cp, needs_layout_passes=False)
    out_type = [jax.ShapeDtypeStruct((NC, NP, DH), jnp.float32)]
    scratch = [
        pltpu.VMEM_SHARED((NP, DH), jnp.float32),
        pltpu.VMEM_SHARED((NP, DH), jnp.float32),
        pltpu.VMEM((2, GS, CH), jnp.int32),
        pltpu.VMEM((2, GS, CH), jnp.int32),
        pltpu.VMEM((CH, DH), jnp.float32),
        pltpu.VMEM((CH, DH), jnp.float32),
    ]
    if with_counts:
        out_type.append(jax.ShapeDtypeStruct((NW, NP), jnp.float32))
        scratch.append(pltpu.VMEM((NP,), jnp.float32))
    scratch.append(pltpu.SemaphoreType.DMA)
    scratch.append(pltpu.SemaphoreType.DMA)
    scratch.append(pltpu.SemaphoreType.DMA)
    return pl.kernel(
        functools.partial(_sc_agg_body, with_counts),
        out_type=tuple(out_type),
        mesh=_sc_mesh,
        scratch_types=scratch,
        compiler_params=cp,
    )


_sc_agg1 = _make_sc_agg(True)
_sc_agg2 = _make_sc_agg(False)


BLK = 1264


def _dot_t(a, w):
    return jax.lax.dot_general(
        a.astype(jnp.bfloat16), w.astype(jnp.bfloat16),
        (((1,), (1,)), ((), ())), preferred_element_type=jnp.float32)


def _tc_layer1_body(part, cntp, x, W1l, b1, W1r, W2l, W2r, b2,
                    p_out, q_out, inv_out):
    agg = jnp.concatenate([part[0], part[1]], axis=1)
    cnt = jnp.sum(cntp[...], axis=0) * 0.5
    inv = 1.0 / jnp.maximum(cnt, 1.0)
    mean = agg * inv[:, None]
    h = _dot_t(mean, W1l[...]) + b1[...] + _dot_t(x[...], W1r[...])
    h = jnp.maximum(h, 0.0)
    p_out[0] = _dot_t(h, W2l[...][:DH])
    p_out[1] = _dot_t(h, W2l[...][DH:])
    q_out[...] = _dot_t(h, W2r[...]) + b2[...]
    inv_out[...] = inv[:, None]


def _tc_layer1(part, cntp, x, W1l, b1, W1r, W2l, b2, W2r):
    return pl.pallas_call(
        _tc_layer1_body,
        out_shape=[
            jax.ShapeDtypeStruct((NC, NP, DH), jnp.float32),
            jax.ShapeDtypeStruct((NP, OUT), jnp.float32),
            jax.ShapeDtypeStruct((NP, 1), jnp.float32),
        ],
    )(part, cntp, x, W1l, b1, W1r, W2l, W2r, b2)


def _tc_final_body(part, inv, q, out):
    agg = jnp.concatenate([part[0, :N], part[1, :N]], axis=1)
    out[...] = agg * inv[...][:N] + q[...][:N]


def _tc_final(part, inv, q):
    return pl.pallas_call(
        _tc_final_body,
        out_shape=jax.ShapeDtypeStruct((N, OUT), jnp.float32),
    )(part, inv, q)


@jax.jit
def _run(x, edge_index, W1l, b1, W1r, W2l, b2, W2r):
    src = edge_index[0]
    dst = edge_index[1]
    srcp = jnp.pad(src, (0, EPAD - E)).reshape(NS, NCHUNK, CH)
    dstp = jnp.pad(dst, (0, EPAD - E),
                   constant_values=N).reshape(NS, NCHUNK, CH)
    xp = jnp.pad(x, ((0, NP - N), (0, 0)))
    x2 = jnp.stack([xp[:, :DH], xp[:, DH:]])

    part1, cntp = _sc_agg1(x2, srcp, dstp)
    p2, q, inv = _tc_layer1(part1, cntp, xp, W1l, b1, W1r, W2l, b2, W2r)
    (part2,) = _sc_agg2(p2, srcp, dstp)
    return _tc_final(part2, inv, q)


def kernel(x, edge_index, W1l, b1, W1r, W2l, b2, W2r):
    return _run(x, edge_index, W1l, b1, W1r, W2l, b2, W2r)
